# baseline (speedup 1.0000x reference)
import functools

import jax
import jax.numpy as jnp
from jax import lax
from jax.experimental import pallas as pl
from jax.experimental.pallas import tpu as pltpu
from jax.experimental.pallas import tpu_sc as plsc

N = 8192
D = 256
K = 8192
TN = 1024
TK = 512
NT = N // TN
KT = K // TK
COMMITMENT_COST = 0.25

_WIN = 4096
_SPLITS = {_WIN // TK: _WIN % TK}


def _minarg(dist, base_col, mask=None):
    if mask is not None:
        dist = jnp.where(mask, dist, jnp.inf)
    tmin = jnp.min(dist, axis=1, keepdims=True)
    kk = lax.broadcasted_iota(jnp.int32, dist.shape, 1) + base_col
    big = jnp.int32(2**31 - 1)
    targ = jnp.min(jnp.where(dist == tmin, kk, big), axis=1, keepdims=True)
    return tmin, targ


def _argmin_body(x_ref, e_ref, x2_ref, e2_ref, idx_out, mind_out,
                 wv, wi, gv, gvraw, gi):
    k = pl.program_id(1)
    d = lax.dot_general(x_ref[...].astype(jnp.bfloat16),
                        e_ref[...].astype(jnp.bfloat16),
                        dimension_numbers=(((1,), (1,)), ((), ())),
                        preferred_element_type=jnp.float32)
    dist = (x2_ref[...] + e2_ref[...]) - 2.0 * d

    def upd_running(tmin, targ):
        u = tmin < wv[...]
        wi[...] = jnp.where(u, targ, wi[...])
        wv[...] = jnp.where(u, tmin, wv[...])

    def close_window():
        take = wv[...] < gv[...]
        gi[...] = jnp.where(take, wi[...], gi[...])
        gvraw[...] = jnp.where(take, wv[...], gvraw[...])
        gv[...] = jnp.where(
            take, wv[...].astype(jnp.bfloat16).astype(jnp.float32), gv[...])
        wv[...] = jnp.full((TN, 1), jnp.inf, jnp.float32)

    @pl.when(k == 0)
    def _():
        gv[...] = jnp.full((TN, 1), jnp.inf, jnp.float32)
        gvraw[...] = jnp.full((TN, 1), jnp.inf, jnp.float32)
        gi[...] = jnp.zeros((TN, 1), jnp.int32)
        tmin, targ = _minarg(dist, 0)
        wv[...] = tmin
        wi[...] = targ

    cols = lax.broadcasted_iota(jnp.int32, (TN, TK), 1)
    for ks, off in _SPLITS.items():
        @pl.when(k == ks)
        def _(ks=ks, off=off):
            if off:
                ta_v, ta_i = _minarg(dist, ks * TK, cols < off)
                upd_running(ta_v, ta_i)
            close_window()
            if off:
                tb_v, tb_i = _minarg(dist, ks * TK, cols >= off)
                upd_running(tb_v, tb_i)
            else:
                t_v, t_i = _minarg(dist, ks * TK)
                upd_running(t_v, t_i)

    is_plain = k > 0
    for ks in _SPLITS:
        is_plain = is_plain & (k != ks)

    @pl.when(is_plain)
    def _():
        tmin, targ = _minarg(dist, k * TK)
        upd_running(tmin, targ)

    @pl.when(k == KT - 1)
    def _():
        take = wv[...] < gv[...]
        idx_out[...] = jnp.where(take, wi[...], gi[...])
        mind_out[...] = jnp.where(take, wv[...], gvraw[...])


def _argmin_indices(flat, emb, x2, e2):
    return pl.pallas_call(
        _argmin_body,
        grid=(NT, KT),
        in_specs=[
            pl.BlockSpec((TN, D), lambda n, k: (n, 0)),
            pl.BlockSpec((TK, D), lambda n, k: (k, 0)),
            pl.BlockSpec((TN, 1), lambda n, k: (n, 0)),
            pl.BlockSpec((1, TK), lambda n, k: (0, k)),
        ],
        out_specs=[
            pl.BlockSpec((TN, 1), lambda n, k: (n, 0)),
            pl.BlockSpec((TN, 1), lambda n, k: (n, 0)),
        ],
        out_shape=[
            jax.ShapeDtypeStruct((N, 1), jnp.int32),
            jax.ShapeDtypeStruct((N, 1), jnp.float32),
        ],
        scratch_shapes=[
            pltpu.VMEM((TN, 1), jnp.float32),
            pltpu.VMEM((TN, 1), jnp.int32),
            pltpu.VMEM((TN, 1), jnp.float32),
            pltpu.VMEM((TN, 1), jnp.float32),
            pltpu.VMEM((TN, 1), jnp.int32),
        ],
    )(flat, emb, x2, e2)


_NC, _NS = 2, 16
_NW = _NC * _NS
_BW = N // _NW
_BINS_W = K // _NS


def _gather_hist(emb, idx):
    mesh = plsc.VectorSubcoreMesh(core_axis_name="c", subcore_axis_name="s")

    @functools.partial(
        pl.kernel,
        mesh=mesh,
        out_type=[
            jax.ShapeDtypeStruct((N, D), jnp.float32),
            jax.ShapeDtypeStruct((_NC, K), jnp.float32),
        ],
        scratch_types=[
            pltpu.VMEM((_BW,), jnp.int32),
            pltpu.VMEM((_BW, D), jnp.float32),
            pltpu.VMEM((_BINS_W,), jnp.float32),
            pltpu.VMEM_SHARED((K,), jnp.float32),
            pltpu.SemaphoreType.DMA,
        ],
    )
    def kern(emb_hbm, idx_hbm, quant_hbm, counts_hbm,
             idx_v, rows_v, stage_v, hist_sh, sem):
        c = lax.axis_index("c")
        s = lax.axis_index("s")
        wid = c * _NS + s
        base = wid * _BW

        pltpu.sync_copy(idx_hbm.at[pl.ds(base, _BW)], idx_v)
        pltpu.async_copy(emb_hbm.at[idx_v], rows_v, sem).wait()
        pltpu.sync_copy(rows_v, quant_hbm.at[pl.ds(base, _BW)])

        z16 = jnp.zeros((16,), jnp.float32)

        def _zero(r, _):
            stage_v[pl.ds(r * 16, 16)] = z16
            return 0
        lax.fori_loop(0, _BINS_W // 16, _zero, 0)
        pltpu.sync_copy(stage_v, hist_sh.at[pl.ds(s * _BINS_W, _BINS_W)])
        plsc.subcore_barrier()

        ones16 = jnp.full((16,), 1.0, jnp.float32)

        def _ones(r, _):
            stage_v[pl.ds(r * 16, 16)] = ones16
            return 0
        lax.fori_loop(0, _BW // 16, _ones, 0)
        pltpu.sync_copy(stage_v.at[pl.ds(0, _BW)], hist_sh.at[idx_v],
                        add=True)
        plsc.subcore_barrier()

        pltpu.sync_copy(hist_sh.at[pl.ds(s * _BINS_W, _BINS_W)],
                        counts_hbm.at[c, pl.ds(s * _BINS_W, _BINS_W)])

    return kern(emb, idx)


def _final_body(mind_ref, ca_ref, cb_ref, loss_ref, perp_ref, probs_ref):
    counts = ca_ref[...] + cb_ref[...]
    p = counts * (1.0 / N)
    probs_ref[...] = p
    ent = jnp.sum(p * jnp.log(p + 1e-10))
    perp_ref[...] = jnp.exp(-ent).reshape(1, 1)
    loss_ref[...] = ((COMMITMENT_COST / (N * D))
                     * jnp.sum(mind_ref[...])).reshape(1, 1)


def _finalize(mind, counts):
    mind2 = mind.reshape(64, 128)
    ca = counts[0].reshape(64, 128)
    cb = counts[1].reshape(64, 128)
    return pl.pallas_call(
        _final_body,
        out_shape=[
            jax.ShapeDtypeStruct((1, 1), jnp.float32),
            jax.ShapeDtypeStruct((1, 1), jnp.float32),
            jax.ShapeDtypeStruct((64, 128), jnp.float32),
        ],
    )(mind2, ca, cb)


def kernel(inputs, embedding):
    input_shape = inputs.shape
    flat = inputs.reshape(-1, D)
    x2 = jnp.sum(flat ** 2, axis=1, keepdims=True)
    e2 = jnp.sum(embedding ** 2, axis=1)

    idx2, mind = _argmin_indices(flat, embedding, x2, e2.reshape(1, K))
    quant, counts = _gather_hist(embedding, idx2.reshape(N))
    loss, perp, probs = _finalize(mind, counts)

    quantized_st = quant.reshape(input_shape)
    return (loss.reshape(()), quantized_st, perp.reshape(()),
            idx2, probs.reshape(K))


# speedup vs baseline: 1.1459x; 1.1459x over previous
import functools

import jax
import jax.numpy as jnp
from jax import lax
from jax.experimental import pallas as pl
from jax.experimental.pallas import tpu as pltpu
from jax.experimental.pallas import tpu_sc as plsc

N = 8192
D = 256
K = 8192
TN = 2048
TK = 512
NT = N // TN
KT = K // TK
COMMITMENT_COST = 0.25

_WIN = 4096
_SPLITS = {_WIN // TK: _WIN % TK}


def _minarg(dist, base_col, mask=None):
    if mask is not None:
        dist = jnp.where(mask, dist, jnp.inf)
    tmin = jnp.min(dist, axis=1, keepdims=True)
    kk = lax.broadcasted_iota(jnp.int32, dist.shape, 1) + base_col
    big = jnp.int32(2**31 - 1)
    targ = jnp.min(jnp.where(dist == tmin, kk, big), axis=1, keepdims=True)
    return tmin, targ


def _argmin_body(x_ref, e_ref, x2_ref, e2_ref, idx_out, mind_out,
                 wv, wi, gv, gvraw, gi):
    k = pl.program_id(1)
    d = lax.dot_general(x_ref[...].astype(jnp.bfloat16),
                        e_ref[...].astype(jnp.bfloat16),
                        dimension_numbers=(((1,), (1,)), ((), ())),
                        preferred_element_type=jnp.float32)
    dist = (x2_ref[...] + e2_ref[...]) - 2.0 * d

    def upd_running(tmin, targ):
        u = tmin < wv[...]
        wi[...] = jnp.where(u, targ, wi[...])
        wv[...] = jnp.where(u, tmin, wv[...])

    def close_window():
        take = wv[...] < gv[...]
        gi[...] = jnp.where(take, wi[...], gi[...])
        gvraw[...] = jnp.where(take, wv[...], gvraw[...])
        gv[...] = jnp.where(
            take, wv[...].astype(jnp.bfloat16).astype(jnp.float32), gv[...])
        wv[...] = jnp.full((TN, 1), jnp.inf, jnp.float32)

    @pl.when(k == 0)
    def _():
        gv[...] = jnp.full((TN, 1), jnp.inf, jnp.float32)
        gvraw[...] = jnp.full((TN, 1), jnp.inf, jnp.float32)
        gi[...] = jnp.zeros((TN, 1), jnp.int32)
        tmin, targ = _minarg(dist, 0)
        wv[...] = tmin
        wi[...] = targ

    cols = lax.broadcasted_iota(jnp.int32, (TN, TK), 1)
    for ks, off in _SPLITS.items():
        @pl.when(k == ks)
        def _(ks=ks, off=off):
            if off:
                ta_v, ta_i = _minarg(dist, ks * TK, cols < off)
                upd_running(ta_v, ta_i)
            close_window()
            if off:
                tb_v, tb_i = _minarg(dist, ks * TK, cols >= off)
                upd_running(tb_v, tb_i)
            else:
                t_v, t_i = _minarg(dist, ks * TK)
                upd_running(t_v, t_i)

    is_plain = k > 0
    for ks in _SPLITS:
        is_plain = is_plain & (k != ks)

    @pl.when(is_plain)
    def _():
        tmin, targ = _minarg(dist, k * TK)
        upd_running(tmin, targ)

    @pl.when(k == KT - 1)
    def _():
        take = wv[...] < gv[...]
        idx_out[...] = jnp.where(take, wi[...], gi[...])
        mind_out[...] = jnp.where(take, wv[...], gvraw[...])


def _argmin_indices(flat, emb, x2, e2):
    return pl.pallas_call(
        _argmin_body,
        grid=(NT, KT),
        in_specs=[
            pl.BlockSpec((TN, D), lambda n, k: (n, 0)),
            pl.BlockSpec((TK, D), lambda n, k: (k, 0)),
            pl.BlockSpec((TN, 1), lambda n, k: (n, 0)),
            pl.BlockSpec((1, TK), lambda n, k: (0, k)),
        ],
        out_specs=[
            pl.BlockSpec((TN, 1), lambda n, k: (n, 0)),
            pl.BlockSpec((TN, 1), lambda n, k: (n, 0)),
        ],
        out_shape=[
            jax.ShapeDtypeStruct((N, 1), jnp.int32),
            jax.ShapeDtypeStruct((N, 1), jnp.float32),
        ],
        scratch_shapes=[
            pltpu.VMEM((TN, 1), jnp.float32),
            pltpu.VMEM((TN, 1), jnp.int32),
            pltpu.VMEM((TN, 1), jnp.float32),
            pltpu.VMEM((TN, 1), jnp.float32),
            pltpu.VMEM((TN, 1), jnp.int32),
        ],
    )(flat, emb, x2, e2)


_NC, _NS = 2, 16
_NW = _NC * _NS
_BW = N // _NW
_BINS_W = K // _NS


def _gather_hist(emb, idx):
    mesh = plsc.VectorSubcoreMesh(core_axis_name="c", subcore_axis_name="s")

    @functools.partial(
        pl.kernel,
        mesh=mesh,
        out_type=[
            jax.ShapeDtypeStruct((N, D), jnp.float32),
            jax.ShapeDtypeStruct((_NC, K), jnp.float32),
        ],
        scratch_types=[
            pltpu.VMEM((_BW,), jnp.int32),
            pltpu.VMEM((_BW, D), jnp.float32),
            pltpu.VMEM((_BINS_W,), jnp.float32),
            pltpu.VMEM_SHARED((K,), jnp.float32),
            pltpu.SemaphoreType.DMA,
        ],
    )
    def kern(emb_hbm, idx_hbm, quant_hbm, counts_hbm,
             idx_v, rows_v, stage_v, hist_sh, sem):
        c = lax.axis_index("c")
        s = lax.axis_index("s")
        wid = c * _NS + s
        base = wid * _BW

        pltpu.sync_copy(idx_hbm.at[pl.ds(base, _BW)], idx_v)
        pltpu.async_copy(emb_hbm.at[idx_v], rows_v, sem).wait()
        pltpu.sync_copy(rows_v, quant_hbm.at[pl.ds(base, _BW)])

        z16 = jnp.zeros((16,), jnp.float32)

        def _zero(r, _):
            stage_v[pl.ds(r * 16, 16)] = z16
            return 0
        lax.fori_loop(0, _BINS_W // 16, _zero, 0)
        pltpu.sync_copy(stage_v, hist_sh.at[pl.ds(s * _BINS_W, _BINS_W)])
        plsc.subcore_barrier()

        ones16 = jnp.full((16,), 1.0, jnp.float32)

        def _ones(r, _):
            stage_v[pl.ds(r * 16, 16)] = ones16
            return 0
        lax.fori_loop(0, _BW // 16, _ones, 0)
        pltpu.sync_copy(stage_v.at[pl.ds(0, _BW)], hist_sh.at[idx_v],
                        add=True)
        plsc.subcore_barrier()

        pltpu.sync_copy(hist_sh.at[pl.ds(s * _BINS_W, _BINS_W)],
                        counts_hbm.at[c, pl.ds(s * _BINS_W, _BINS_W)])

    return kern(emb, idx)


def _final_body(mind_ref, ca_ref, cb_ref, loss_ref, perp_ref, probs_ref):
    counts = ca_ref[...] + cb_ref[...]
    p = counts * (1.0 / N)
    probs_ref[...] = p
    ent = jnp.sum(p * jnp.log(p + 1e-10))
    perp_ref[...] = jnp.exp(-ent).reshape(1, 1)
    loss_ref[...] = ((COMMITMENT_COST / (N * D))
                     * jnp.sum(mind_ref[...])).reshape(1, 1)


def _finalize(mind, counts):
    mind2 = mind.reshape(64, 128)
    ca = counts[0].reshape(64, 128)
    cb = counts[1].reshape(64, 128)
    return pl.pallas_call(
        _final_body,
        out_shape=[
            jax.ShapeDtypeStruct((1, 1), jnp.float32),
            jax.ShapeDtypeStruct((1, 1), jnp.float32),
            jax.ShapeDtypeStruct((64, 128), jnp.float32),
        ],
    )(mind2, ca, cb)


def kernel(inputs, embedding):
    input_shape = inputs.shape
    flat = inputs.reshape(-1, D)
    x2 = jnp.sum(flat ** 2, axis=1, keepdims=True)
    e2 = jnp.sum(embedding ** 2, axis=1)

    idx2, mind = _argmin_indices(flat, embedding, x2, e2.reshape(1, K))
    quant, counts = _gather_hist(embedding, idx2.reshape(N))
    loss, perp, probs = _finalize(mind, counts)

    quantized_st = quant.reshape(input_shape)
    return (loss.reshape(()), quantized_st, perp.reshape(()),
            idx2, probs.reshape(K))
